# baseline (speedup 1.0000x reference)
import functools

import jax
import jax.numpy as jnp
from jax import lax
from jax.experimental import pallas as pl
from jax.experimental.pallas import tpu as pltpu
from jax.experimental.pallas import tpu_sc as plsc

N_NODES = 100000
BATCH = 10000
S = 16
D = 128

NC = 2
NS = 16
NW = NC * NS
BP = 10240
BPW = BP // NW
CROWS = 8
NCHUNK = BPW // CROWS
RED = 6240
REDTAIL = N_NODES - RED * NS
NGRP = BPW // 16


def _worker(base_only=False):
    cid = lax.axis_index("c")
    sid = lax.axis_index("s")
    wid = sid * NC + cid
    return cid, sid, wid, wid * BPW


def _dw_body(nodes2d, neigh2d, rc, rowv, clumv, outscale,
             nidx2, gidx2, rcn_v, rcnb_v, scale_v, redbuf, tailbuf,
             pbuf, allbuf, shared, semi, semn, semc0, semc1):
    cid, sid, wid, base = _worker()

    d_nidx = pltpu.make_async_copy(nodes2d.at[pl.ds(wid * 8, 8)], nidx2, semi)
    d_gidx = pltpu.make_async_copy(
        neigh2d.at[pl.ds(wid * NCHUNK, NCHUNK)], gidx2, semi)
    d_nidx.start()
    d_gidx.start()

    with jax.named_scope("maxred"):
        neg = jnp.full((16,), -3.0e38, jnp.float32)

        def _reduce_slice(src):
            pltpu.sync_copy(src.at[pl.ds(sid * RED, RED)], redbuf)
            pltpu.sync_copy(src.at[pl.ds(NS * RED, REDTAIL)], tailbuf)

            def body(i, m):
                for k in range(10):
                    m = jnp.maximum(m, redbuf[pl.ds(i * 160 + k * 16, 16)])
                return m

            m = lax.fori_loop(0, RED // 160, body, neg)
            for k in range(REDTAIL // 16):
                m = jnp.maximum(m, tailbuf[pl.ds(k * 16, 16)])
            return m

        mrow = _reduce_slice(rowv)
        mclum = _reduce_slice(clumv)

        pbuf[0, :] = mrow
        pbuf[1, :] = mclum
        pltpu.sync_copy(pbuf, shared.at[sid])
        plsc.subcore_barrier()
        pltpu.sync_copy(shared, allbuf)
        for t in range(NS):
            mrow = jnp.maximum(mrow, allbuf[t, 0, :])
            mclum = jnp.maximum(mclum, allbuf[t, 1, :])

        il = lax.iota(jnp.int32, 16)

        def _lane_max(v):
            for sh in (8, 4, 2, 1):
                tailbuf[pl.ds(0, 16)] = v
                v = jnp.maximum(v, plsc.load_gather(tailbuf, [il ^ sh]))
            return v

        inv_b = 1.0 / _lane_max(mrow)
        inv_a = 1.0 / _lane_max(mclum)

    d_nidx.wait()
    d_gidx.wait()

    ndescs = []
    for t in range(8):
        dsc = pltpu.make_async_copy(
            rc.at[nidx2.at[t]], rcn_v.at[pl.ds(t * 40, 40)], semn)
        dsc.start()
        ndescs.append(dsc)

    zero16 = jnp.zeros((16,), jnp.int32)
    one16 = zero16 + 1
    semc = (semc0, semc1)

    def coord_descs(g, p):
        return (
            pltpu.make_async_copy(
                rc.at[gidx2.at[2 * g]], rcnb_v.at[p, pl.ds(0, 128)], semc[p]),
            pltpu.make_async_copy(
                rc.at[gidx2.at[2 * g + 1]], rcnb_v.at[p, pl.ds(128, 128)],
                semc[p]),
        )

    def grp_valid(g):
        return base + g * 16 < BATCH

    for g in range(2):
        for dsc in coord_descs(g, g):
            dsc.start()

    for dd in ndescs:
        dd.wait()

    def dw_group_inner(g, p):
        for dsc in coord_descs(g, p):
            dsc.wait()

        @pl.when(jnp.logical_and(g + 2 < NGRP, grp_valid(g + 2)))
        def _fire():
            for dsc in coord_descs(g + 2, p):
                dsc.start()

        rbase = g * 16 + il
        rown = plsc.load_gather(rcn_v, [rbase, zero16]) * inv_b
        clumn = plsc.load_gather(rcn_v, [rbase, one16]) * inv_a
        srow = jnp.zeros((16,), jnp.float32)
        sclum = jnp.zeros((16,), jnp.float32)
        nb = il * S
        for j in range(S):
            srow = srow + plsc.load_gather(rcnb_v.at[p], [nb + j, zero16])
            sclum = sclum + plsc.load_gather(rcnb_v.at[p], [nb + j, one16])
        row_sum = srow * (1.0 / S) * inv_b
        clum_sum = sclum * (1.0 / S) * inv_a
        dr = row_sum - rown
        dc = clum_sum - clumn
        d2 = dr * dr + dc * dc + 1e-12
        dw = 1.0 / (1.0 + jnp.exp(-1.0 / d2))
        scale_v[pl.ds(g * 16, 16)] = dw * (1.0 / S)

    def dw_group(g, p):
        @pl.when(grp_valid(g))
        def _():
            dw_group_inner(g, p)

    def dw_outer(t, carry):
        for p in range(2):
            dw_group(t * 2 + p, p)
        return carry

    with jax.named_scope("dweight"):
        lax.fori_loop(0, NGRP // 2, dw_outer, 0)

    pltpu.sync_copy(scale_v, outscale.at[pl.ds(base, BPW)])


def _main_body(neigh2d, feat, scale_in, out,
               gidx2, sc8, obuf, gbuf0, gbuf1, gbuf2, gbuf3,
               semi, semf0, semf1, semf2, semf3, semo0, semo1):
    cid, sid, wid, base = _worker()

    d_gidx = pltpu.make_async_copy(
        neigh2d.at[pl.ds(wid * NCHUNK, NCHUNK)], gidx2, semi)
    d_scale = pltpu.make_async_copy(scale_in.at[pl.ds(wid * 8, 8)], sc8, semi)
    d_gidx.start()
    d_scale.start()
    d_gidx.wait()
    d_scale.wait()

    gb = (gbuf0, gbuf1, gbuf2, gbuf3)
    semf = (semf0, semf1, semf2, semf3)
    semo = (semo0, semo1)

    def feat_desc(c, p):
        return pltpu.make_async_copy(feat.at[gidx2.at[c]], gb[p], semf[p])

    def chunk_valid(c):
        return base + c * CROWS < BATCH

    for c in range(3):
        feat_desc(c, c).start()

    def store_desc(c, po):
        s = base + c * CROWS
        return pltpu.make_async_copy(
            obuf.at[po], out.at[pl.ds(s, CROWS)], semo[po])

    def chunk_compute(gbuf, c, po):
        def row_body(r, carry):
            accs = [gbuf[r * S, pl.ds(k * 16, 16)] for k in range(D // 16)]
            for j in range(1, S):
                for k in range(D // 16):
                    accs[k] = accs[k] + gbuf[r * S + j, pl.ds(k * 16, 16)]
            widx = jnp.zeros((16,), jnp.int32) + (c * CROWS + r)
            q = widx // 128
            w2 = 2.0 * plsc.load_gather(sc8, [q, widx - q * 128])
            for k in range(D // 16):
                e = jnp.exp(w2 * accs[k])
                obuf[po, r, pl.ds(k * 16, 16)] = (e - 1.0) / (e + 1.0)
            return carry

        lax.fori_loop(0, CROWS, row_body, 0)

    def outer(t, carry):
        for p in range(4):
            c = t * 4 + p
            po = p % 2

            @pl.when(chunk_valid(c))
            def _chunk():
                feat_desc(c, p).wait()

                @pl.when(jnp.logical_and(c + 3 < NCHUNK, chunk_valid(c + 3)))
                def _fire():
                    feat_desc(c + 3, (p + 3) % 4).start()

                @pl.when(c >= 2)
                def _drain():
                    store_desc(c - 2, po).wait()

                chunk_compute(gb[p], c, po)
                store_desc(c, po).start()
        return carry

    with jax.named_scope("mainloop"):
        lax.fori_loop(0, NCHUNK // 4, outer, 0)

    store_desc(0, 0).wait()
    store_desc(1, 1).wait()


@jax.jit
def kernel(nodes, neigh_idx, features, row, clum):
    nodes2d = jnp.pad(nodes, (0, BP - BATCH)).reshape(BP // 40, 40)
    neigh_p = jnp.pad(neigh_idx.reshape(-1), (0, (BP - BATCH) * S))
    neigh2d = neigh_p.reshape(BP * S // 128, 128)
    rc = jnp.concatenate(
        [row[:, None], clum[:, None],
         jnp.zeros((N_NODES, 14), jnp.float32)], axis=1)

    mesh = plsc.VectorSubcoreMesh(core_axis_name="c", subcore_axis_name="s")

    dw_kernel = functools.partial(
        pl.kernel,
        out_type=jax.ShapeDtypeStruct((BP,), jnp.float32),
        mesh=mesh,
        compiler_params=pltpu.CompilerParams(
            needs_layout_passes=False, use_tc_tiling_on_sc=False),
        scratch_types=[
            pltpu.VMEM((8, 40), jnp.int32),
            pltpu.VMEM((NCHUNK, 128), jnp.int32),
            pltpu.VMEM((BPW, 16), jnp.float32),
            pltpu.VMEM((2, 256, 16), jnp.float32),
            pltpu.VMEM((BPW,), jnp.float32),
            pltpu.VMEM((RED,), jnp.float32),
            pltpu.VMEM((REDTAIL,), jnp.float32),
            pltpu.VMEM((2, 16), jnp.float32),
            pltpu.VMEM((NS, 2, 16), jnp.float32),
            pltpu.VMEM_SHARED((NS, 2, 16), jnp.float32),
            pltpu.SemaphoreType.DMA,
            pltpu.SemaphoreType.DMA,
            pltpu.SemaphoreType.DMA,
            pltpu.SemaphoreType.DMA,
        ],
    )(_dw_body)
    scale = dw_kernel(nodes2d, neigh2d, rc, row, clum)

    scale_a = jnp.pad(scale.reshape(NW, BPW), ((0, 0), (0, 1024 - BPW)))
    scale_a = scale_a.reshape(NW * 8, 128)

    main_kernel = functools.partial(
        pl.kernel,
        out_type=jax.ShapeDtypeStruct((BATCH, D), jnp.float32),
        mesh=mesh,
        compiler_params=pltpu.CompilerParams(
            needs_layout_passes=False, use_tc_tiling_on_sc=True),
        scratch_types=[
            pltpu.VMEM((NCHUNK, 128), jnp.int32),
            pltpu.VMEM((8, 128), jnp.float32),
            pltpu.VMEM((2, CROWS, D), jnp.float32),
            pltpu.VMEM((128, D), jnp.float32),
            pltpu.VMEM((128, D), jnp.float32),
            pltpu.VMEM((128, D), jnp.float32),
            pltpu.VMEM((128, D), jnp.float32),
            pltpu.SemaphoreType.DMA,
            pltpu.SemaphoreType.DMA,
            pltpu.SemaphoreType.DMA,
            pltpu.SemaphoreType.DMA,
            pltpu.SemaphoreType.DMA,
            pltpu.SemaphoreType.DMA,
            pltpu.SemaphoreType.DMA,
        ],
    )(_main_body)
    return main_kernel(neigh2d, features, scale_a)


# speedup vs baseline: 1.3929x; 1.3929x over previous
import functools

import jax
import jax.numpy as jnp
from jax import lax
from jax.experimental import pallas as pl
from jax.experimental.pallas import tpu as pltpu
from jax.experimental.pallas import tpu_sc as plsc

N_NODES = 100000
BATCH = 10000
S = 16
D = 128

NC = 2
NS = 16
NW = NC * NS
BP = 10240
BPW = BP // NW
CROWS = 8
NCHUNK = BPW // CROWS
RED = 6240
REDTAIL = N_NODES - RED * NS
NGRP = BPW // 16


def _worker(base_only=False):
    cid = lax.axis_index("c")
    sid = lax.axis_index("s")
    wid = sid * NC + cid
    return cid, sid, wid, wid * BPW


def _dw_body(nodes2d, neigh2d, rowv, clumv, outscale, rcout,
             nidx2, gidx2, rcn_v, rcnb_v, scale_v, redbuf, redbuf2,
             tailbuf, tailbuf2, pbuf, allbuf, bld, shared,
             semi, semn, semc0, semc1, semb):
    cid, sid, wid, base = _worker()

    d_nidx = pltpu.make_async_copy(nodes2d.at[pl.ds(wid * 8, 8)], nidx2, semi)
    d_gidx = pltpu.make_async_copy(
        neigh2d.at[pl.ds(wid * NCHUNK, NCHUNK)], gidx2, semi)
    d_nidx.start()
    d_gidx.start()

    il = lax.iota(jnp.int32, 16)
    zero16 = jnp.zeros((16,), jnp.int32)
    one16 = zero16 + 1

    pltpu.sync_copy(rowv.at[pl.ds(sid * RED, RED)], redbuf)
    pltpu.sync_copy(clumv.at[pl.ds(sid * RED, RED)], redbuf2)
    pltpu.sync_copy(rowv.at[pl.ds(NS * RED, REDTAIL)], tailbuf)
    pltpu.sync_copy(clumv.at[pl.ds(NS * RED, REDTAIL)], tailbuf2)

    with jax.named_scope("rcbuild"):
        BLD = 624

        def bld_chunk(q, p, n, src_r, src_c, dst_off):
            def bi(i, carry):
                lanes = i * 16 + il
                plsc.store_scatter(
                    bld.at[p], [lanes, zero16], src_r[pl.ds(q * BLD + i * 16, 16)])
                plsc.store_scatter(
                    bld.at[p], [lanes, one16], src_c[pl.ds(q * BLD + i * 16, 16)])
                return carry

            lax.fori_loop(0, n // 16, bi, 0)
            return pltpu.make_async_copy(
                bld.at[p, pl.ds(0, n)], rcout.at[pl.ds(dst_off, n)], semb)

        pend = [None, None]
        for q in range(RED // BLD):
            p = q % 2
            if pend[p] is not None:
                pend[p].wait()
            d = bld_chunk(q, p, BLD, redbuf, redbuf2, sid * RED + q * BLD)
            d.start()
            pend[p] = d
        pend[0].wait()
        pend[1].wait()

        @pl.when(sid == 0)
        def _tail_build():
            d = bld_chunk(0, 0, REDTAIL, tailbuf, tailbuf2, NS * RED)
            d.start()
            d.wait()

    with jax.named_scope("maxred"):
        neg = jnp.full((16,), -3.0e38, jnp.float32)

        def _reduce_buf(buf, tbuf):
            def body(i, m):
                for k in range(10):
                    m = jnp.maximum(m, buf[pl.ds(i * 160 + k * 16, 16)])
                return m

            m = lax.fori_loop(0, RED // 160, body, neg)
            for k in range(REDTAIL // 16):
                m = jnp.maximum(m, tbuf[pl.ds(k * 16, 16)])
            return m

        mrow = _reduce_buf(redbuf, tailbuf)
        mclum = _reduce_buf(redbuf2, tailbuf2)

        pbuf[0, :] = mrow
        pbuf[1, :] = mclum
        pltpu.sync_copy(pbuf, shared.at[sid])
        plsc.subcore_barrier()
        pltpu.sync_copy(shared, allbuf)
        for t in range(NS):
            mrow = jnp.maximum(mrow, allbuf[t, 0, :])
            mclum = jnp.maximum(mclum, allbuf[t, 1, :])

        il = lax.iota(jnp.int32, 16)

        def _lane_max(v):
            for sh in (8, 4, 2, 1):
                tailbuf[pl.ds(0, 16)] = v
                v = jnp.maximum(v, plsc.load_gather(tailbuf, [il ^ sh]))
            return v

        inv_b = 1.0 / _lane_max(mrow)
        inv_a = 1.0 / _lane_max(mclum)

    d_nidx.wait()
    d_gidx.wait()

    ndescs = []
    for t in range(8):
        dsc = pltpu.make_async_copy(
            rcout.at[nidx2.at[t]], rcn_v.at[pl.ds(t * 40, 40)], semn)
        dsc.start()
        ndescs.append(dsc)

    zero16 = jnp.zeros((16,), jnp.int32)
    one16 = zero16 + 1
    semc = (semc0, semc1)

    def coord_descs(g, p):
        return (
            pltpu.make_async_copy(
                rcout.at[gidx2.at[2 * g]], rcnb_v.at[p, pl.ds(0, 128)], semc[p]),
            pltpu.make_async_copy(
                rcout.at[gidx2.at[2 * g + 1]], rcnb_v.at[p, pl.ds(128, 128)],
                semc[p]),
        )

    def grp_valid(g):
        return base + g * 16 < BATCH

    for g in range(2):
        for dsc in coord_descs(g, g):
            dsc.start()

    for dd in ndescs:
        dd.wait()

    def dw_group_inner(g, p):
        for dsc in coord_descs(g, p):
            dsc.wait()

        @pl.when(jnp.logical_and(g + 2 < NGRP, grp_valid(g + 2)))
        def _fire():
            for dsc in coord_descs(g + 2, p):
                dsc.start()

        rbase = g * 16 + il
        rown = plsc.load_gather(rcn_v, [rbase, zero16]) * inv_b
        clumn = plsc.load_gather(rcn_v, [rbase, one16]) * inv_a
        srow = jnp.zeros((16,), jnp.float32)
        sclum = jnp.zeros((16,), jnp.float32)
        nb = il * S
        for j in range(S):
            srow = srow + plsc.load_gather(rcnb_v.at[p], [nb + j, zero16])
            sclum = sclum + plsc.load_gather(rcnb_v.at[p], [nb + j, one16])
        row_sum = srow * (1.0 / S) * inv_b
        clum_sum = sclum * (1.0 / S) * inv_a
        dr = row_sum - rown
        dc = clum_sum - clumn
        d2 = dr * dr + dc * dc + 1e-12
        dw = 1.0 / (1.0 + jnp.exp(-1.0 / d2))
        scale_v[pl.ds(g * 16, 16)] = dw * (1.0 / S)

    def dw_group(g, p):
        @pl.when(grp_valid(g))
        def _():
            dw_group_inner(g, p)

    def dw_outer(t, carry):
        for p in range(2):
            dw_group(t * 2 + p, p)
        return carry

    with jax.named_scope("dweight"):
        lax.fori_loop(0, NGRP // 2, dw_outer, 0)

    pltpu.sync_copy(scale_v, outscale.at[pl.ds(base, BPW)])


def _main_body(neigh2d, feat, scale_in, out,
               gidx2, sc8, obuf, gbuf0, gbuf1, gbuf2, gbuf3,
               semi, semf0, semf1, semf2, semf3, semo0, semo1):
    cid, sid, wid, base = _worker()

    d_gidx = pltpu.make_async_copy(
        neigh2d.at[pl.ds(wid * NCHUNK, NCHUNK)], gidx2, semi)
    d_scale = pltpu.make_async_copy(scale_in.at[pl.ds(wid * 8, 8)], sc8, semi)
    d_gidx.start()
    d_scale.start()
    d_gidx.wait()
    d_scale.wait()

    gb = (gbuf0, gbuf1, gbuf2, gbuf3)
    semf = (semf0, semf1, semf2, semf3)
    semo = (semo0, semo1)

    def feat_desc(c, p):
        return pltpu.make_async_copy(feat.at[gidx2.at[c]], gb[p], semf[p])

    def chunk_valid(c):
        return base + c * CROWS < BATCH

    for c in range(3):
        feat_desc(c, c).start()

    def store_desc(c, po):
        s = base + c * CROWS
        return pltpu.make_async_copy(
            obuf.at[po], out.at[pl.ds(s, CROWS)], semo[po])

    def chunk_compute(gbuf, c, po):
        def row_body(r, carry):
            accs = [gbuf[r * S, pl.ds(k * 16, 16)] for k in range(D // 16)]
            for j in range(1, S):
                for k in range(D // 16):
                    accs[k] = accs[k] + gbuf[r * S + j, pl.ds(k * 16, 16)]
            widx = jnp.zeros((16,), jnp.int32) + (c * CROWS + r)
            q = widx // 128
            w2 = 2.0 * plsc.load_gather(sc8, [q, widx - q * 128])
            for k in range(D // 16):
                e = jnp.exp(w2 * accs[k])
                obuf[po, r, pl.ds(k * 16, 16)] = (e - 1.0) / (e + 1.0)
            return carry

        lax.fori_loop(0, CROWS, row_body, 0)

    def outer(t, carry):
        for p in range(4):
            c = t * 4 + p
            po = p % 2

            @pl.when(chunk_valid(c))
            def _chunk():
                feat_desc(c, p).wait()

                @pl.when(jnp.logical_and(c + 3 < NCHUNK, chunk_valid(c + 3)))
                def _fire():
                    feat_desc(c + 3, (p + 3) % 4).start()

                @pl.when(c >= 2)
                def _drain():
                    store_desc(c - 2, po).wait()

                chunk_compute(gb[p], c, po)
                store_desc(c, po).start()
        return carry

    with jax.named_scope("mainloop"):
        lax.fori_loop(0, NCHUNK // 4, outer, 0)

    store_desc(0, 0).wait()
    store_desc(1, 1).wait()


@jax.jit
def kernel(nodes, neigh_idx, features, row, clum):
    nodes2d = jnp.pad(nodes, (0, BP - BATCH)).reshape(BP // 40, 40)
    neigh_p = jnp.pad(neigh_idx.reshape(-1), (0, (BP - BATCH) * S))
    neigh2d = neigh_p.reshape(BP * S // 128, 128)

    mesh = plsc.VectorSubcoreMesh(core_axis_name="c", subcore_axis_name="s")

    dw_kernel = functools.partial(
        pl.kernel,
        out_type=(jax.ShapeDtypeStruct((BP,), jnp.float32),
                  jax.ShapeDtypeStruct((N_NODES, 16), jnp.float32)),
        mesh=mesh,
        compiler_params=pltpu.CompilerParams(
            needs_layout_passes=False, use_tc_tiling_on_sc=False),
        scratch_types=[
            pltpu.VMEM((8, 40), jnp.int32),
            pltpu.VMEM((NCHUNK, 128), jnp.int32),
            pltpu.VMEM((BPW, 16), jnp.float32),
            pltpu.VMEM((2, 256, 16), jnp.float32),
            pltpu.VMEM((BPW,), jnp.float32),
            pltpu.VMEM((RED,), jnp.float32),
            pltpu.VMEM((RED,), jnp.float32),
            pltpu.VMEM((REDTAIL,), jnp.float32),
            pltpu.VMEM((REDTAIL,), jnp.float32),
            pltpu.VMEM((2, 16), jnp.float32),
            pltpu.VMEM((NS, 2, 16), jnp.float32),
            pltpu.VMEM((2, 624, 16), jnp.float32),
            pltpu.VMEM_SHARED((NS, 2, 16), jnp.float32),
            pltpu.SemaphoreType.DMA,
            pltpu.SemaphoreType.DMA,
            pltpu.SemaphoreType.DMA,
            pltpu.SemaphoreType.DMA,
            pltpu.SemaphoreType.DMA,
        ],
    )(_dw_body)
    scale, _rc_unused = dw_kernel(nodes2d, neigh2d, row, clum)

    scale_a = jnp.pad(scale.reshape(NW, BPW), ((0, 0), (0, 1024 - BPW)))
    scale_a = scale_a.reshape(NW * 8, 128)

    main_kernel = functools.partial(
        pl.kernel,
        out_type=jax.ShapeDtypeStruct((BATCH, D), jnp.float32),
        mesh=mesh,
        compiler_params=pltpu.CompilerParams(
            needs_layout_passes=False, use_tc_tiling_on_sc=True),
        scratch_types=[
            pltpu.VMEM((NCHUNK, 128), jnp.int32),
            pltpu.VMEM((8, 128), jnp.float32),
            pltpu.VMEM((2, CROWS, D), jnp.float32),
            pltpu.VMEM((128, D), jnp.float32),
            pltpu.VMEM((128, D), jnp.float32),
            pltpu.VMEM((128, D), jnp.float32),
            pltpu.VMEM((128, D), jnp.float32),
            pltpu.SemaphoreType.DMA,
            pltpu.SemaphoreType.DMA,
            pltpu.SemaphoreType.DMA,
            pltpu.SemaphoreType.DMA,
            pltpu.SemaphoreType.DMA,
            pltpu.SemaphoreType.DMA,
            pltpu.SemaphoreType.DMA,
        ],
    )(_main_body)
    return main_kernel(neigh2d, features, scale_a)


# speedup vs baseline: 1.4416x; 1.0350x over previous
import functools

import jax
import jax.numpy as jnp
from jax import lax
from jax.experimental import pallas as pl
from jax.experimental.pallas import tpu as pltpu
from jax.experimental.pallas import tpu_sc as plsc

N_NODES = 100000
BATCH = 10000
S = 16
D = 128

NC = 2
NS = 16
NW = NC * NS
BP = 10240
BPW = BP // NW
CROWS = 8
NCHUNK = BPW // CROWS
RED = 6240
REDTAIL = N_NODES - RED * NS
NGRP = BPW // 16


def _worker(base_only=False):
    cid = lax.axis_index("c")
    sid = lax.axis_index("s")
    wid = sid * NC + cid
    return cid, sid, wid, wid * BPW


def _dw_body(nodes2d, neigh2d, rowv, clumv, outscale, rcout,
             nidx2, gidx2, rcn_v, rcnb_v, scale_v, redbuf, redbuf2,
             tailbuf, tailbuf2, pbuf, allbuf, bld, shared,
             semi, semn, semc0, semc1, semc2, semc3, semb):
    cid, sid, wid, base = _worker()

    d_nidx = pltpu.make_async_copy(nodes2d.at[pl.ds(wid * 8, 8)], nidx2, semi)
    d_gidx = pltpu.make_async_copy(
        neigh2d.at[pl.ds(wid * NCHUNK, NCHUNK)], gidx2, semi)
    d_nidx.start()
    d_gidx.start()

    il = lax.iota(jnp.int32, 16)
    zero16 = jnp.zeros((16,), jnp.int32)
    one16 = zero16 + 1

    pltpu.sync_copy(rowv.at[pl.ds(sid * RED, RED)], redbuf)
    pltpu.sync_copy(clumv.at[pl.ds(sid * RED, RED)], redbuf2)
    pltpu.sync_copy(rowv.at[pl.ds(NS * RED, REDTAIL)], tailbuf)
    pltpu.sync_copy(clumv.at[pl.ds(NS * RED, REDTAIL)], tailbuf2)

    with jax.named_scope("rcbuild"):
        BLD = 624

        def bld_chunk(q, p, n, src_r, src_c, dst_off):
            def bi(i, carry):
                lanes = i * 16 + il
                plsc.store_scatter(
                    bld.at[p], [lanes, zero16], src_r[pl.ds(q * BLD + i * 16, 16)])
                plsc.store_scatter(
                    bld.at[p], [lanes, one16], src_c[pl.ds(q * BLD + i * 16, 16)])
                return carry

            lax.fori_loop(0, n // 16, bi, 0)
            return pltpu.make_async_copy(
                bld.at[p, pl.ds(0, n)], rcout.at[pl.ds(dst_off, n)], semb)

        pend = [None, None]
        for q in range(RED // BLD):
            p = q % 2
            if pend[p] is not None:
                pend[p].wait()
            d = bld_chunk(q, p, BLD, redbuf, redbuf2, sid * RED + q * BLD)
            d.start()
            pend[p] = d
        pend[0].wait()
        pend[1].wait()

        @pl.when(sid == 0)
        def _tail_build():
            d = bld_chunk(0, 0, REDTAIL, tailbuf, tailbuf2, NS * RED)
            d.start()
            d.wait()

    with jax.named_scope("maxred"):
        neg = jnp.full((16,), -3.0e38, jnp.float32)

        def _reduce_buf(buf, tbuf):
            def body(i, m):
                for k in range(10):
                    m = jnp.maximum(m, buf[pl.ds(i * 160 + k * 16, 16)])
                return m

            m = lax.fori_loop(0, RED // 160, body, neg)
            for k in range(REDTAIL // 16):
                m = jnp.maximum(m, tbuf[pl.ds(k * 16, 16)])
            return m

        mrow = _reduce_buf(redbuf, tailbuf)
        mclum = _reduce_buf(redbuf2, tailbuf2)

        pbuf[0, :] = mrow
        pbuf[1, :] = mclum
        pltpu.sync_copy(pbuf, shared.at[sid])
        plsc.subcore_barrier()
        pltpu.sync_copy(shared, allbuf)
        for t in range(NS):
            mrow = jnp.maximum(mrow, allbuf[t, 0, :])
            mclum = jnp.maximum(mclum, allbuf[t, 1, :])

        il = lax.iota(jnp.int32, 16)

        def _lane_max(v):
            for sh in (8, 4, 2, 1):
                tailbuf[pl.ds(0, 16)] = v
                v = jnp.maximum(v, plsc.load_gather(tailbuf, [il ^ sh]))
            return v

        inv_b = 1.0 / _lane_max(mrow)
        inv_a = 1.0 / _lane_max(mclum)

    d_nidx.wait()
    d_gidx.wait()

    ndescs = []
    for t in range(8):
        dsc = pltpu.make_async_copy(
            rcout.at[nidx2.at[t]], rcn_v.at[pl.ds(t * 40, 40)], semn)
        dsc.start()
        ndescs.append(dsc)

    semc = (semc0, semc1, semc2, semc3)

    def coord_descs(g, p):
        return (
            pltpu.make_async_copy(
                rcout.at[gidx2.at[2 * g]], rcnb_v.at[p, pl.ds(0, 128)], semc[p]),
            pltpu.make_async_copy(
                rcout.at[gidx2.at[2 * g + 1]], rcnb_v.at[p, pl.ds(128, 128)],
                semc[p]),
        )

    def grp_valid(g):
        return base + g * 16 < BATCH

    for g in range(4):
        for dsc in coord_descs(g, g):
            dsc.start()

    for dd in ndescs:
        dd.wait()

    def dw_group_inner(g, p):
        for dsc in coord_descs(g, p):
            dsc.wait()

        @pl.when(jnp.logical_and(g + 4 < NGRP, grp_valid(g + 4)))
        def _fire():
            for dsc in coord_descs(g + 4, p):
                dsc.start()

        rbase = g * 16 + il
        rown = plsc.load_gather(rcn_v, [rbase, zero16]) * inv_b
        clumn = plsc.load_gather(rcn_v, [rbase, one16]) * inv_a
        srow = jnp.zeros((16,), jnp.float32)
        sclum = jnp.zeros((16,), jnp.float32)
        nb = il * S
        for j in range(S):
            srow = srow + plsc.load_gather(rcnb_v.at[p], [nb + j, zero16])
            sclum = sclum + plsc.load_gather(rcnb_v.at[p], [nb + j, one16])
        row_sum = srow * (1.0 / S) * inv_b
        clum_sum = sclum * (1.0 / S) * inv_a
        dr = row_sum - rown
        dc = clum_sum - clumn
        d2 = dr * dr + dc * dc + 1e-12
        dw = 1.0 / (1.0 + jnp.exp(-1.0 / d2))
        scale_v[g // 8, pl.ds((g % 8) * 16, 16)] = dw * (1.0 / S)

    def dw_group(g, p):
        @pl.when(grp_valid(g))
        def _():
            dw_group_inner(g, p)

    def dw_outer(t, carry):
        for p in range(4):
            dw_group(t * 4 + p, p)
        return carry

    with jax.named_scope("dweight"):
        lax.fori_loop(0, NGRP // 4, dw_outer, 0)

    pltpu.sync_copy(scale_v, outscale.at[pl.ds(wid * 8, 8)])


def _main_body(neigh2d, feat, scale_in, out,
               gidx2, sc8, obuf, gbuf0, gbuf1, gbuf2, gbuf3,
               semi, semf0, semf1, semf2, semf3, semo0, semo1):
    cid, sid, wid, base = _worker()

    d_gidx = pltpu.make_async_copy(
        neigh2d.at[pl.ds(wid * NCHUNK, NCHUNK)], gidx2, semi)
    d_scale = pltpu.make_async_copy(scale_in.at[pl.ds(wid * 8, 8)], sc8, semi)
    d_gidx.start()
    d_scale.start()
    d_gidx.wait()
    d_scale.wait()

    gb = (gbuf0, gbuf1, gbuf2, gbuf3)
    semf = (semf0, semf1, semf2, semf3)
    semo = (semo0, semo1)

    def feat_desc(c, p):
        return pltpu.make_async_copy(feat.at[gidx2.at[c]], gb[p], semf[p])

    def chunk_valid(c):
        return base + c * CROWS < BATCH

    for c in range(3):
        feat_desc(c, c).start()

    def store_desc(c, po):
        s = base + c * CROWS
        return pltpu.make_async_copy(
            obuf.at[po], out.at[pl.ds(s, CROWS)], semo[po])

    def chunk_compute(gbuf, c, po):
        def one_row(r):
            accs = [gbuf[r * S, pl.ds(k * 16, 16)] for k in range(D // 16)]
            for j in range(1, S):
                for k in range(D // 16):
                    accs[k] = accs[k] + gbuf[r * S + j, pl.ds(k * 16, 16)]
            widx = jnp.zeros((16,), jnp.int32) + (c * CROWS + r)
            q = widx // 128
            w2 = 2.0 * plsc.load_gather(sc8, [q, widx - q * 128])
            for k in range(D // 16):
                e = jnp.exp(w2 * accs[k])
                obuf[po, r, pl.ds(k * 16, 16)] = (e - 1.0) / (e + 1.0)

        def row_body(h, carry):
            one_row(h * 2)
            one_row(h * 2 + 1)
            return carry

        lax.fori_loop(0, CROWS // 2, row_body, 0)

    def outer(t, carry):
        for p in range(4):
            c = t * 4 + p
            po = p % 2

            @pl.when(chunk_valid(c))
            def _chunk():
                feat_desc(c, p).wait()

                @pl.when(jnp.logical_and(c + 3 < NCHUNK, chunk_valid(c + 3)))
                def _fire():
                    feat_desc(c + 3, (p + 3) % 4).start()

                @pl.when(c >= 2)
                def _drain():
                    store_desc(c - 2, po).wait()

                chunk_compute(gb[p], c, po)
                store_desc(c, po).start()
        return carry

    with jax.named_scope("mainloop"):
        lax.fori_loop(0, NCHUNK // 4, outer, 0)

    store_desc(0, 0).wait()
    store_desc(1, 1).wait()


@jax.jit
def kernel(nodes, neigh_idx, features, row, clum):
    nodes2d = jnp.pad(nodes, (0, BP - BATCH)).reshape(BP // 40, 40)
    neigh_p = jnp.pad(neigh_idx.reshape(-1), (0, (BP - BATCH) * S))
    neigh2d = neigh_p.reshape(BP * S // 128, 128)

    mesh = plsc.VectorSubcoreMesh(core_axis_name="c", subcore_axis_name="s")

    dw_kernel = functools.partial(
        pl.kernel,
        out_type=(jax.ShapeDtypeStruct((NW * 8, 128), jnp.float32),
                  jax.ShapeDtypeStruct((N_NODES, 16), jnp.float32)),
        mesh=mesh,
        compiler_params=pltpu.CompilerParams(
            needs_layout_passes=False, use_tc_tiling_on_sc=False),
        scratch_types=[
            pltpu.VMEM((8, 40), jnp.int32),
            pltpu.VMEM((NCHUNK, 128), jnp.int32),
            pltpu.VMEM((BPW, 16), jnp.float32),
            pltpu.VMEM((4, 256, 16), jnp.float32),
            pltpu.VMEM((8, 128), jnp.float32),
            pltpu.VMEM((RED,), jnp.float32),
            pltpu.VMEM((RED,), jnp.float32),
            pltpu.VMEM((REDTAIL,), jnp.float32),
            pltpu.VMEM((REDTAIL,), jnp.float32),
            pltpu.VMEM((2, 16), jnp.float32),
            pltpu.VMEM((NS, 2, 16), jnp.float32),
            pltpu.VMEM((2, 624, 16), jnp.float32),
            pltpu.VMEM_SHARED((NS, 2, 16), jnp.float32),
            pltpu.SemaphoreType.DMA,
            pltpu.SemaphoreType.DMA,
            pltpu.SemaphoreType.DMA,
            pltpu.SemaphoreType.DMA,
            pltpu.SemaphoreType.DMA,
            pltpu.SemaphoreType.DMA,
            pltpu.SemaphoreType.DMA,
        ],
    )(_dw_body)
    scale_a, _rc_unused = dw_kernel(nodes2d, neigh2d, row, clum)

    main_kernel = functools.partial(
        pl.kernel,
        out_type=jax.ShapeDtypeStruct((BATCH, D), jnp.float32),
        mesh=mesh,
        compiler_params=pltpu.CompilerParams(
            needs_layout_passes=False, use_tc_tiling_on_sc=True),
        scratch_types=[
            pltpu.VMEM((NCHUNK, 128), jnp.int32),
            pltpu.VMEM((8, 128), jnp.float32),
            pltpu.VMEM((2, CROWS, D), jnp.float32),
            pltpu.VMEM((128, D), jnp.float32),
            pltpu.VMEM((128, D), jnp.float32),
            pltpu.VMEM((128, D), jnp.float32),
            pltpu.VMEM((128, D), jnp.float32),
            pltpu.SemaphoreType.DMA,
            pltpu.SemaphoreType.DMA,
            pltpu.SemaphoreType.DMA,
            pltpu.SemaphoreType.DMA,
            pltpu.SemaphoreType.DMA,
            pltpu.SemaphoreType.DMA,
            pltpu.SemaphoreType.DMA,
        ],
    )(_main_body)
    return main_kernel(neigh2d, features, scale_a)


# speedup vs baseline: 1.4566x; 1.0104x over previous
import functools

import jax
import jax.numpy as jnp
from jax import lax
from jax.experimental import pallas as pl
from jax.experimental.pallas import tpu as pltpu
from jax.experimental.pallas import tpu_sc as plsc

N_NODES = 100000
BATCH = 10000
S = 16
D = 128

NC = 2
NS = 16
NW = NC * NS
BP = 10240
BPW = BP // NW
CROWS = 8
NCHUNK = BPW // CROWS
RED = 6240
REDTAIL = N_NODES - RED * NS
NGRP = BPW // 16
GIDX_ROWS = NCHUNK


def _worker(base_only=False):
    cid = lax.axis_index("c")
    sid = lax.axis_index("s")
    wid = sid * NC + cid
    return cid, sid, wid, wid * BPW


def _dw_body(nodes2d, neigh2d, rowv, clumv, outscale, rcout,
             nidx2, gidx2, rcn_v, rcnb_v, scale_v, redbuf, redbuf2,
             tailbuf, tailbuf2, pbuf, allbuf, bld, shared,
             semi, semn, semc0, semc1, semc2, semc3, semb):
    cid, sid, wid, base = _worker()

    goff = wid * NCHUNK
    dgc = 0
    noff = jnp.minimum(wid * 8, BATCH // 40 - 8)
    dn = (wid * 8 - noff) * 40
    d_nidx = pltpu.make_async_copy(nodes2d.at[pl.ds(noff, 8)], nidx2, semi)
    d_gidx = pltpu.make_async_copy(
        neigh2d.at[pl.ds(goff, GIDX_ROWS)], gidx2, semi)
    d_nidx.start()
    d_gidx.start()

    il = lax.iota(jnp.int32, 16)
    zero16 = jnp.zeros((16,), jnp.int32)
    one16 = zero16 + 1

    pltpu.sync_copy(rowv.at[pl.ds(sid * RED, RED)], redbuf)
    pltpu.sync_copy(clumv.at[pl.ds(sid * RED, RED)], redbuf2)
    pltpu.sync_copy(rowv.at[pl.ds(NS * RED, REDTAIL)], tailbuf)
    pltpu.sync_copy(clumv.at[pl.ds(NS * RED, REDTAIL)], tailbuf2)

    with jax.named_scope("rcbuild"):
        BLD = 624

        def bld_chunk(q, p, n, src_r, src_c, dst_off):
            def bi(i, carry):
                lanes = i * 16 + il
                plsc.store_scatter(
                    bld.at[p], [lanes, zero16], src_r[pl.ds(q * BLD + i * 16, 16)])
                plsc.store_scatter(
                    bld.at[p], [lanes, one16], src_c[pl.ds(q * BLD + i * 16, 16)])
                return carry

            lax.fori_loop(0, n // 16, bi, 0)
            return pltpu.make_async_copy(
                bld.at[p, pl.ds(0, n)], rcout.at[pl.ds(dst_off, n)], semb)

        pend = [None, None]
        for q in range(RED // BLD):
            p = q % 2
            if pend[p] is not None:
                pend[p].wait()
            d = bld_chunk(q, p, BLD, redbuf, redbuf2, sid * RED + q * BLD)
            d.start()
            pend[p] = d
        pend[0].wait()
        pend[1].wait()

        @pl.when(sid == 0)
        def _tail_build():
            d = bld_chunk(0, 0, REDTAIL, tailbuf, tailbuf2, NS * RED)
            d.start()
            d.wait()

    with jax.named_scope("maxred"):
        neg = jnp.full((16,), -3.0e38, jnp.float32)

        def _reduce_buf(buf, tbuf):
            def body(i, m):
                for k in range(10):
                    m = jnp.maximum(m, buf[pl.ds(i * 160 + k * 16, 16)])
                return m

            m = lax.fori_loop(0, RED // 160, body, neg)
            for k in range(REDTAIL // 16):
                m = jnp.maximum(m, tbuf[pl.ds(k * 16, 16)])
            return m

        mrow = _reduce_buf(redbuf, tailbuf)
        mclum = _reduce_buf(redbuf2, tailbuf2)

        pbuf[0, :] = mrow
        pbuf[1, :] = mclum
        pltpu.sync_copy(pbuf, shared.at[sid])
        plsc.subcore_barrier()
        pltpu.sync_copy(shared, allbuf)
        for t in range(NS):
            mrow = jnp.maximum(mrow, allbuf[t, 0, :])
            mclum = jnp.maximum(mclum, allbuf[t, 1, :])

        il = lax.iota(jnp.int32, 16)

        def _lane_max(v):
            for sh in (8, 4, 2, 1):
                tailbuf[pl.ds(0, 16)] = v
                v = jnp.maximum(v, plsc.load_gather(tailbuf, [il ^ sh]))
            return v

        inv_b = 1.0 / _lane_max(mrow)
        inv_a = 1.0 / _lane_max(mclum)

    d_nidx.wait()
    d_gidx.wait()

    ndescs = []
    for t in range(8):
        dsc = pltpu.make_async_copy(
            rcout.at[nidx2.at[t]], rcn_v.at[pl.ds(t * 40, 40)], semn)
        dsc.start()
        ndescs.append(dsc)

    semc = (semc0, semc1, semc2, semc3)

    def coord_descs(g, p):
        return (
            pltpu.make_async_copy(
                rcout.at[gidx2.at[2 * g + dgc]], rcnb_v.at[p, pl.ds(0, 128)],
                semc[p]),
            pltpu.make_async_copy(
                rcout.at[gidx2.at[2 * g + 1 + dgc]], rcnb_v.at[p, pl.ds(128, 128)],
                semc[p]),
        )

    def grp_valid(g):
        return base + g * 16 < BATCH

    for g in range(4):
        for dsc in coord_descs(g, g):
            dsc.start()

    for dd in ndescs:
        dd.wait()

    def dw_group_inner(g, p):
        for dsc in coord_descs(g, p):
            dsc.wait()

        @pl.when(jnp.logical_and(g + 4 < NGRP, grp_valid(g + 4)))
        def _fire():
            for dsc in coord_descs(g + 4, p):
                dsc.start()

        rbase = g * 16 + il + dn
        rown = plsc.load_gather(rcn_v, [rbase, zero16]) * inv_b
        clumn = plsc.load_gather(rcn_v, [rbase, one16]) * inv_a
        srow = jnp.zeros((16,), jnp.float32)
        sclum = jnp.zeros((16,), jnp.float32)
        nb = il * S
        for j in range(S):
            srow = srow + plsc.load_gather(rcnb_v.at[p], [nb + j, zero16])
            sclum = sclum + plsc.load_gather(rcnb_v.at[p], [nb + j, one16])
        row_sum = srow * (1.0 / S) * inv_b
        clum_sum = sclum * (1.0 / S) * inv_a
        dr = row_sum - rown
        dc = clum_sum - clumn
        d2 = dr * dr + dc * dc + 1e-12
        dw = 1.0 / (1.0 + jnp.exp(-1.0 / d2))
        scale_v[g // 8, pl.ds((g % 8) * 16, 16)] = dw * (1.0 / S)

    def dw_group(g, p):
        @pl.when(grp_valid(g))
        def _():
            dw_group_inner(g, p)

    def dw_outer(t, carry):
        for p in range(4):
            dw_group(t * 4 + p, p)
        return carry

    with jax.named_scope("dweight"):
        lax.fori_loop(0, NGRP // 4, dw_outer, 0)

    pltpu.sync_copy(scale_v, outscale.at[pl.ds(wid * 8, 8)])


def _main_body(neigh2d, feat, scale_in, out,
               gidx2, sc8, obuf, gbuf0, gbuf1, gbuf2, gbuf3,
               semi, semf0, semf1, semf2, semf3, semo0, semo1):
    cid, sid, wid, base = _worker()

    goff = wid * NCHUNK
    dgc = 0
    d_gidx = pltpu.make_async_copy(
        neigh2d.at[pl.ds(goff, GIDX_ROWS)], gidx2, semi)
    d_scale = pltpu.make_async_copy(scale_in.at[pl.ds(wid * 8, 8)], sc8, semi)
    d_gidx.start()
    d_scale.start()
    d_gidx.wait()
    d_scale.wait()

    gb = (gbuf0, gbuf1, gbuf2, gbuf3)
    semf = (semf0, semf1, semf2, semf3)
    semo = (semo0, semo1)

    def feat_desc(c, p):
        return pltpu.make_async_copy(
            feat.at[gidx2.at[c + dgc]], gb[p], semf[p])

    def chunk_valid(c):
        return base + c * CROWS < BATCH

    for c in range(3):
        feat_desc(c, c).start()

    def store_desc(c, po):
        s = base + c * CROWS
        return pltpu.make_async_copy(
            obuf.at[po], out.at[pl.ds(s, CROWS)], semo[po])

    def chunk_compute(gbuf, c, po):
        def one_row(r):
            accs = [gbuf[r * S, pl.ds(k * 16, 16)] for k in range(D // 16)]
            for j in range(1, S):
                for k in range(D // 16):
                    accs[k] = accs[k] + gbuf[r * S + j, pl.ds(k * 16, 16)]
            widx = jnp.zeros((16,), jnp.int32) + (c * CROWS + r)
            q = widx // 128
            w2 = 2.0 * plsc.load_gather(sc8, [q, widx - q * 128])
            for k in range(D // 16):
                e = jnp.exp(w2 * accs[k])
                obuf[po, r, pl.ds(k * 16, 16)] = (e - 1.0) / (e + 1.0)

        def row_body(h, carry):
            one_row(h * 2)
            one_row(h * 2 + 1)
            return carry

        lax.fori_loop(0, CROWS // 2, row_body, 0)

    def outer(t, carry):
        for p in range(4):
            c = t * 4 + p
            po = p % 2

            @pl.when(chunk_valid(c))
            def _chunk():
                feat_desc(c, p).wait()

                @pl.when(jnp.logical_and(c + 3 < NCHUNK, chunk_valid(c + 3)))
                def _fire():
                    feat_desc(c + 3, (p + 3) % 4).start()

                @pl.when(c >= 2)
                def _drain():
                    store_desc(c - 2, po).wait()

                chunk_compute(gb[p], c, po)
                store_desc(c, po).start()
        return carry

    with jax.named_scope("mainloop"):
        lax.fori_loop(0, NCHUNK // 4, outer, 0)

    store_desc(0, 0).wait()
    store_desc(1, 1).wait()


@jax.jit
def kernel(nodes, neigh_idx, features, row, clum):
    nodes2d = nodes.reshape(BATCH // 40, 40)
    neigh2d = jnp.pad(neigh_idx.reshape(BATCH * S // 128, 128),
                      ((0, BP * S // 128 - BATCH * S // 128), (0, 0)))

    mesh = plsc.VectorSubcoreMesh(core_axis_name="c", subcore_axis_name="s")

    dw_kernel = functools.partial(
        pl.kernel,
        out_type=(jax.ShapeDtypeStruct((NW * 8, 128), jnp.float32),
                  jax.ShapeDtypeStruct((N_NODES, 16), jnp.float32)),
        mesh=mesh,
        compiler_params=pltpu.CompilerParams(
            needs_layout_passes=False, use_tc_tiling_on_sc=False),
        scratch_types=[
            pltpu.VMEM((8, 40), jnp.int32),
            pltpu.VMEM((GIDX_ROWS, 128), jnp.int32),
            pltpu.VMEM((BPW, 16), jnp.float32),
            pltpu.VMEM((4, 256, 16), jnp.float32),
            pltpu.VMEM((8, 128), jnp.float32),
            pltpu.VMEM((RED,), jnp.float32),
            pltpu.VMEM((RED,), jnp.float32),
            pltpu.VMEM((REDTAIL,), jnp.float32),
            pltpu.VMEM((REDTAIL,), jnp.float32),
            pltpu.VMEM((2, 16), jnp.float32),
            pltpu.VMEM((NS, 2, 16), jnp.float32),
            pltpu.VMEM((2, 624, 16), jnp.float32),
            pltpu.VMEM_SHARED((NS, 2, 16), jnp.float32),
            pltpu.SemaphoreType.DMA,
            pltpu.SemaphoreType.DMA,
            pltpu.SemaphoreType.DMA,
            pltpu.SemaphoreType.DMA,
            pltpu.SemaphoreType.DMA,
            pltpu.SemaphoreType.DMA,
            pltpu.SemaphoreType.DMA,
        ],
    )(_dw_body)
    scale_a, _rc_unused = dw_kernel(nodes2d, neigh2d, row, clum)

    main_kernel = functools.partial(
        pl.kernel,
        out_type=jax.ShapeDtypeStruct((BATCH, D), jnp.float32),
        mesh=mesh,
        compiler_params=pltpu.CompilerParams(
            needs_layout_passes=False, use_tc_tiling_on_sc=True),
        scratch_types=[
            pltpu.VMEM((GIDX_ROWS, 128), jnp.int32),
            pltpu.VMEM((8, 128), jnp.float32),
            pltpu.VMEM((2, CROWS, D), jnp.float32),
            pltpu.VMEM((128, D), jnp.float32),
            pltpu.VMEM((128, D), jnp.float32),
            pltpu.VMEM((128, D), jnp.float32),
            pltpu.VMEM((128, D), jnp.float32),
            pltpu.SemaphoreType.DMA,
            pltpu.SemaphoreType.DMA,
            pltpu.SemaphoreType.DMA,
            pltpu.SemaphoreType.DMA,
            pltpu.SemaphoreType.DMA,
            pltpu.SemaphoreType.DMA,
            pltpu.SemaphoreType.DMA,
        ],
    )(_main_body)
    return main_kernel(neigh2d, features, scale_a)
